# Initial kernel scaffold; baseline (speedup 1.0000x reference)
import jax
import jax.numpy as jnp
from jax.experimental import pallas as pl


def kernel(x):
    raise NotImplementedError("write your pallas kernel here")


# speedup vs baseline: 9.6194x; 9.6194x over previous
import jax
import jax.numpy as jnp
import numpy as np
from jax.experimental import pallas as pl

_NUM = 16
_H = 224
_W = 224
_BLK = 16


def _pools():
    kx = jax.random.fold_in(jax.random.key(1), 0)
    ky = jax.random.fold_in(jax.random.key(1), 1)
    pool_x = np.asarray(jax.random.permutation(kx, _H))
    pool_y = np.asarray(jax.random.permutation(ky, _W))
    return pool_x, pool_y


_POOL_X, _POOL_Y = _pools()
_PX = [int(v) for v in _POOL_X[:_NUM]]
_PY = [int(v) for v in _POOL_Y[:_NUM]]
_SX = [int(v) for v in _POOL_X[-_NUM:]]
_SY = [int(v) for v in _POOL_Y[-_NUM:]]


def _copy_fixup_kernel(x_ref, o_ref):
    o_ref[...] = x_ref[...]
    col = jax.lax.broadcasted_iota(jnp.int32, (_BLK, _W), 1)
    for px, py, sx, sy in zip(_PX, _PY, _SX, _SY):
        src_rows = x_ref[:, sx, :]
        vals = jnp.sum(jnp.where(col == sy, src_rows, 0.0), axis=1, keepdims=True)
        dst_rows = x_ref[:, px, :]
        o_ref[:, px, :] = jnp.where(col == py, vals, dst_rows)


def kernel(x):
    b, c, h, w = x.shape
    xr = x.reshape(b * c, h, w)
    out = pl.pallas_call(
        _copy_fixup_kernel,
        out_shape=jax.ShapeDtypeStruct(xr.shape, xr.dtype),
        grid=(b * c // _BLK,),
        in_specs=[pl.BlockSpec((_BLK, h, w), lambda i: (i, 0, 0))],
        out_specs=pl.BlockSpec((_BLK, h, w), lambda i: (i, 0, 0)),
    )(xr)
    loc = (jnp.asarray(_POOL_X[:_NUM]), jnp.asarray(_POOL_Y[:_NUM]))
    return (out.reshape(b, c, h, w), loc)


# speedup vs baseline: 9.6274x; 1.0008x over previous
import jax
import jax.numpy as jnp
import numpy as np
from jax.experimental import pallas as pl
from jax.experimental.pallas import tpu as pltpu

_NUM = 16
_H = 224
_W = 224
_BLK = 16


def _pools():
    kx = jax.random.fold_in(jax.random.key(1), 0)
    ky = jax.random.fold_in(jax.random.key(1), 1)
    pool_x = np.asarray(jax.random.permutation(kx, _H))
    pool_y = np.asarray(jax.random.permutation(ky, _W))
    return pool_x, pool_y


_POOL_X, _POOL_Y = _pools()
_PX = [int(v) for v in _POOL_X[:_NUM]]
_PY = [int(v) for v in _POOL_Y[:_NUM]]
_SX = [int(v) for v in _POOL_X[-_NUM:]]
_SY = [int(v) for v in _POOL_Y[-_NUM:]]


def _copy_fixup_kernel(x_ref, o_ref):
    o_ref[...] = x_ref[...]
    col = jax.lax.broadcasted_iota(jnp.int32, (_BLK, _W), 1)
    for px, py, sx, sy in zip(_PX, _PY, _SX, _SY):
        src_rows = x_ref[:, sx, :]
        vals = jnp.sum(jnp.where(col == sy, src_rows, 0.0), axis=1, keepdims=True)
        dst_rows = x_ref[:, px, :]
        o_ref[:, px, :] = jnp.where(col == py, vals, dst_rows)


def kernel(x):
    b, c, h, w = x.shape
    xr = x.reshape(b * c, h, w)
    out = pl.pallas_call(
        _copy_fixup_kernel,
        out_shape=jax.ShapeDtypeStruct(xr.shape, xr.dtype),
        grid=(b * c // _BLK,),
        in_specs=[pl.BlockSpec((_BLK, h, w), lambda i: (i, 0, 0))],
        out_specs=pl.BlockSpec((_BLK, h, w), lambda i: (i, 0, 0)),
        compiler_params=pltpu.CompilerParams(
            dimension_semantics=("parallel",),
        ),
    )(xr)
    loc = (jnp.asarray(_POOL_X[:_NUM]), jnp.asarray(_POOL_Y[:_NUM]))
    return (out.reshape(b, c, h, w), loc)


# speedup vs baseline: 9.8507x; 1.0232x over previous
import jax
import jax.numpy as jnp
import numpy as np
from jax.experimental import pallas as pl
from jax.experimental.pallas import tpu as pltpu

_NUM = 16
_H = 224
_W = 224
_BLK = 32


def _pools():
    kx = jax.random.fold_in(jax.random.key(1), 0)
    ky = jax.random.fold_in(jax.random.key(1), 1)
    pool_x = np.asarray(jax.random.permutation(kx, _H))
    pool_y = np.asarray(jax.random.permutation(ky, _W))
    return pool_x, pool_y


_POOL_X, _POOL_Y = _pools()
_PX = [int(v) for v in _POOL_X[:_NUM]]
_PY = [int(v) for v in _POOL_Y[:_NUM]]
_SX = [int(v) for v in _POOL_X[-_NUM:]]
_SY = [int(v) for v in _POOL_Y[-_NUM:]]


def _copy_fixup_kernel(x_ref, o_ref):
    o_ref[...] = x_ref[...]
    col = jax.lax.broadcasted_iota(jnp.int32, (_BLK, _W), 1)
    for px, py, sx, sy in zip(_PX, _PY, _SX, _SY):
        src_rows = x_ref[:, sx, :]
        vals = jnp.sum(jnp.where(col == sy, src_rows, 0.0), axis=1, keepdims=True)
        dst_rows = x_ref[:, px, :]
        o_ref[:, px, :] = jnp.where(col == py, vals, dst_rows)


def kernel(x):
    b, c, h, w = x.shape
    xr = x.reshape(b * c, h, w)
    out = pl.pallas_call(
        _copy_fixup_kernel,
        out_shape=jax.ShapeDtypeStruct(xr.shape, xr.dtype),
        grid=(b * c // _BLK,),
        in_specs=[pl.BlockSpec((_BLK, h, w), lambda i: (i, 0, 0))],
        out_specs=pl.BlockSpec((_BLK, h, w), lambda i: (i, 0, 0)),
        compiler_params=pltpu.CompilerParams(
            dimension_semantics=("parallel",),
        ),
    )(xr)
    loc = (jnp.asarray(_POOL_X[:_NUM]), jnp.asarray(_POOL_Y[:_NUM]))
    return (out.reshape(b, c, h, w), loc)


# speedup vs baseline: 9.9686x; 1.0120x over previous
import jax
import jax.numpy as jnp
import numpy as np
from jax.experimental import pallas as pl
from jax.experimental.pallas import tpu as pltpu

_NUM = 16
_H = 224
_W = 224
_BLK = 64


def _pools():
    kx = jax.random.fold_in(jax.random.key(1), 0)
    ky = jax.random.fold_in(jax.random.key(1), 1)
    pool_x = np.asarray(jax.random.permutation(kx, _H))
    pool_y = np.asarray(jax.random.permutation(ky, _W))
    return pool_x, pool_y


_POOL_X, _POOL_Y = _pools()
_PX = [int(v) for v in _POOL_X[:_NUM]]
_PY = [int(v) for v in _POOL_Y[:_NUM]]
_SX = [int(v) for v in _POOL_X[-_NUM:]]
_SY = [int(v) for v in _POOL_Y[-_NUM:]]


def _copy_fixup_kernel(x_ref, o_ref):
    o_ref[...] = x_ref[...]
    col = jax.lax.broadcasted_iota(jnp.int32, (_BLK, _W), 1)
    for px, py, sx, sy in zip(_PX, _PY, _SX, _SY):
        src_rows = x_ref[:, sx, :]
        vals = jnp.sum(jnp.where(col == sy, src_rows, 0.0), axis=1, keepdims=True)
        dst_rows = x_ref[:, px, :]
        o_ref[:, px, :] = jnp.where(col == py, vals, dst_rows)


def kernel(x):
    b, c, h, w = x.shape
    xr = x.reshape(b * c, h, w)
    out = pl.pallas_call(
        _copy_fixup_kernel,
        out_shape=jax.ShapeDtypeStruct(xr.shape, xr.dtype),
        grid=(b * c // _BLK,),
        in_specs=[pl.BlockSpec((_BLK, h, w), lambda i: (i, 0, 0))],
        out_specs=pl.BlockSpec((_BLK, h, w), lambda i: (i, 0, 0)),
        compiler_params=pltpu.CompilerParams(
            dimension_semantics=("parallel",),
        ),
    )(xr)
    loc = (jnp.asarray(_POOL_X[:_NUM]), jnp.asarray(_POOL_Y[:_NUM]))
    return (out.reshape(b, c, h, w), loc)
